# Initial kernel scaffold; baseline (speedup 1.0000x reference)
import jax
import jax.numpy as jnp
from jax.experimental import pallas as pl


def kernel(x, edge_index, edge_attr, batch, eps, edge_W, edge_b, W1, b1, g1, be1, W2, b2, bng, bnb, head_W, head_b):
    raise NotImplementedError("write your pallas kernel here")


# speedup vs baseline: 1.5406x; 1.5406x over previous
import functools

import jax
import jax.numpy as jnp
from jax import lax
from jax.experimental import pallas as pl
from jax.experimental.pallas import tpu as pltpu
from jax.experimental.pallas import tpu_sc as plsc

N = 10000
E = 320000
D = 128
DE = 16
L = 5
G = 128
T = 128
H = 2 * D

_NC = 2
_NS = 16
_NW = _NC * _NS
_SH = E // _NW
_WCH = 128
_SHP = 10112
_NCH = _SHP // _WCH
_SIDE = N
_DUMMY = N + 64
_RPT = 632
_NPAD = _NS * _RPT


def _edge_embed(edge_attr, w, b):
    BE = 4000

    def body(a_ref, w_ref, b_ref, o_ref):
        o_ref[...] = (
            jnp.dot(a_ref[...], w_ref[...], preferred_element_type=jnp.float32)
            + b_ref[...]
        )

    return pl.pallas_call(
        body,
        grid=(E // BE,),
        in_specs=[
            pl.BlockSpec((BE, DE), lambda i: (i, 0)),
            pl.BlockSpec((DE, D), lambda i: (0, 0)),
            pl.BlockSpec((1, D), lambda i: (0, 0)),
        ],
        out_specs=pl.BlockSpec((BE, D), lambda i: (i, 0)),
        out_shape=jax.ShapeDtypeStruct((E, D), jnp.float32),
    )(edge_attr, w, b[None, :])


def _edge_plan(src, dst):
    perm = jnp.argsort(dst, stable=True)
    ds = dst[perm]
    srcs = src[perm]
    pos = jnp.arange(E, dtype=jnp.int32)
    w = pos // _SH
    shard_first = ds[w * _SH]
    shard_last = ds[w * _SH + (_SH - 1)]
    nxt = jnp.concatenate([ds[1:], jnp.full((1,), -1, jnp.int32)])
    last_of_shard = (pos % _SH) == (_SH - 1)
    fin = last_of_shard | (ds != nxt)
    firstseg = ds == shard_first
    lastseg = ds == shard_last
    tgt = jnp.where(
        ~fin, _DUMMY,
        jnp.where(lastseg, _SIDE + 2 * w + 1,
                  jnp.where(firstseg, _SIDE + 2 * w, ds)))
    prv = jnp.concatenate([jnp.full((1,), -1, jnp.int32), ds[:-1]])
    keep = jnp.where(((pos % _SH) == 0) | (ds != prv), 0.0, 1.0
                     ).astype(jnp.float32)

    def pad(a, fill):
        a2 = a.reshape(_NW, _SH)
        p = jnp.full((_NW, _SHP - _SH), fill, a.dtype)
        return jnp.concatenate([a2, p], axis=1).reshape(-1)

    perm_p = pad(perm.astype(jnp.int32), 0)
    srcs_p = pad(srcs, 0)
    tgt_p = pad(tgt.astype(jnp.int32), _DUMMY)
    keep_p = pad(keep, 0.0)
    keep_p = jnp.repeat(keep_p[:, None], 16, axis=1)

    first_d = ds[jnp.arange(_NW) * _SH]
    last_d = ds[jnp.arange(_NW) * _SH + (_SH - 1)]
    multi = first_d != last_d
    ids0 = jnp.where(multi, first_d, -1)
    ids1 = last_d
    ids = jnp.stack([ids0, ids1], axis=1).reshape(-1)
    valid = ids >= 0

    def scanf(carry, x):
        i, v = x
        return jnp.where(v, i, carry), carry

    _, prevv = lax.scan(scanf, jnp.int32(-2), (ids, valid))
    start = (valid & (ids != prevv)).astype(jnp.int32)
    use = valid.astype(jnp.float32)
    _, nxtv = lax.scan(scanf, jnp.int32(-2),
                       (jnp.flip(ids), jnp.flip(valid)))
    nxtv = jnp.flip(nxtv)
    wrow = jnp.where(valid & (ids != nxtv), ids, -1).astype(jnp.int32)
    core_of = (jnp.arange(64, dtype=jnp.int32) // 2) // _NS
    return perm_p, srcs_p, tgt_p, keep_p, start, use, wrow, core_of


def _sc_edge(h, e, perm_p, srcs_p, tgt_p, keep_p):
    mesh = plsc.VectorSubcoreMesh(core_axis_name="c", subcore_axis_name="s")

    @functools.partial(
        pl.kernel,
        out_type=jax.ShapeDtypeStruct((_NC, _NPAD, D), jnp.float32),
        mesh=mesh,
        scratch_types=[
            pltpu.VMEM((1, 128), jnp.int32),
            pltpu.VMEM((1, 128), jnp.int32),
            pltpu.VMEM((1, 128), jnp.int32),
            pltpu.VMEM((_WCH, 16), jnp.float32),
            pltpu.VMEM((_WCH, D), jnp.float32),
            pltpu.VMEM((_WCH, D), jnp.float32),
            pltpu.VMEM((1, D), jnp.float32),
            pltpu.VMEM_SHARED((_NPAD, D), jnp.float32),
            pltpu.SemaphoreType.DMA,
        ],
    )
    def sc_k(h_hbm, e_hbm, perm_hbm, srcs_hbm, tgt_hbm, keep_hbm, out_hbm,
             pidx, sidx, tidx, kv, e_v, g_v, acc_v, agg_sh, sem):
        c = lax.axis_index("c")
        s = lax.axis_index("s")
        shard = c * _NS + s
        base = shard * _SHP

        @pl.loop(0, _WCH)
        def _zrow(r):
            for k in range(D // 16):
                e_v[r, pl.ds(k * 16, 16)] = jnp.zeros((16,), jnp.float32)

        row0 = s * _RPT
        nfull = _RPT // _WCH
        rem = _RPT - nfull * _WCH
        for t in range(nfull):
            pltpu.sync_copy(e_v, agg_sh.at[pl.ds(row0 + t * _WCH, _WCH)])
        pltpu.sync_copy(e_v.at[pl.ds(0, rem)],
                        agg_sh.at[pl.ds(row0 + nfull * _WCH, rem)])
        for k in range(D // 16):
            acc_v[0, pl.ds(k * 16, 16)] = jnp.zeros((16,), jnp.float32)
        plsc.subcore_barrier()

        @pl.loop(0, _NCH)
        def _chunk(ci):
            eb = base + ci * _WCH
            pltpu.sync_copy(perm_hbm.at[pl.ds(eb, _WCH)], pidx.at[0])
            pltpu.sync_copy(srcs_hbm.at[pl.ds(eb, _WCH)], sidx.at[0])
            pltpu.sync_copy(tgt_hbm.at[pl.ds(eb, _WCH)], tidx.at[0])
            pltpu.sync_copy(keep_hbm.at[pl.ds(eb, _WCH)], kv)
            pltpu.async_copy(e_hbm.at[pidx.at[0]], e_v, sem).wait()
            pltpu.async_copy(h_hbm.at[sidx.at[0]], g_v, sem).wait()

            def edge_body(r, acc):
                kf = kv[r, pl.ds(0, 16)]
                new = []
                for k in range(D // 16):
                    sl = pl.ds(k * 16, 16)
                    m = jnp.maximum(e_v[r, sl] + g_v[r, sl], 0.0)
                    a = acc[k] * kf + m
                    e_v[r, sl] = a
                    new.append(a)
                return tuple(new)

            acc0 = tuple(acc_v[0, pl.ds(k * 16, 16)] for k in range(D // 16))
            accn = lax.fori_loop(0, _WCH, edge_body, acc0)
            for k in range(D // 16):
                acc_v[0, pl.ds(k * 16, 16)] = accn[k]

            pltpu.sync_copy(e_v, agg_sh.at[tidx.at[0]])

        plsc.subcore_barrier()

        for t in range(nfull):
            r0 = row0 + t * _WCH
            pltpu.sync_copy(agg_sh.at[pl.ds(r0, _WCH)], e_v)
            pltpu.sync_copy(e_v, out_hbm.at[c, pl.ds(r0, _WCH)])
        r0 = row0 + nfull * _WCH
        pltpu.sync_copy(agg_sh.at[pl.ds(r0, rem)], e_v.at[pl.ds(0, rem)])
        pltpu.sync_copy(e_v.at[pl.ds(0, rem)], out_hbm.at[c, pl.ds(r0, rem)])

    return sc_k(h, e, perm_p, srcs_p, tgt_p, keep_p)


def _stats(y_ref, mu_ref, va_ref, W):
    nch = N // 8

    def macc(ci, acc):
        return acc + y_ref[pl.ds(ci * 8, 8), :]

    acc = lax.fori_loop(0, nch, macc, jnp.zeros((8, W), jnp.float32))
    b = acc[0:4] + acc[4:8]
    c2 = b[0:2] + b[2:4]
    mu = (c2[0:1] + c2[1:2]) * jnp.float32(1e-4)
    mu_ref[...] = mu

    def vacc(ci, acc):
        d8 = y_ref[pl.ds(ci * 8, 8), :] - mu
        return acc + d8 * d8

    a1 = lax.fori_loop(0, nch // 2, vacc, jnp.zeros((8, W), jnp.float32))
    a2 = lax.fori_loop(nch // 2, nch, vacc, jnp.zeros((8, W), jnp.float32))

    def tree(acc):
        b = acc[0:4] + acc[4:8]
        c2 = b[0:2] + b[2:4]
        return c2[0:1] + c2[1:2]

    va_ref[...] = (tree(a1) + tree(a2)) * jnp.float32(1e-4)


def _stage_a(h, parts, eps_l, w1, b1, start, use, wrow, core_of):
    def body(h_ref, p_ref, eps_ref, w1_ref, b1_ref, start_ref, use_ref,
             wrow_ref, core_ref, o_ref, mu_ref, va_ref, z_ref):
        z_ref[pl.ds(0, N), :] = ((1.0 + eps_ref[0]) * h_ref[...]
                                 + p_ref[0, :N, :] + p_ref[1, :N, :])

        def mrg(k, acc):
            sv0 = p_ref[0, pl.ds(N + k, 1), :]
            sv1 = p_ref[1, pl.ds(N + k, 1), :]
            sv = jnp.where(core_ref[k] == 0, sv0, sv1) * use_ref[k]
            acc = jnp.where(start_ref[k] == 1, sv, acc + sv)
            row = wrow_ref[k]
            trow = jnp.where(row >= 0, row, N)
            cur = z_ref[pl.ds(trow, 1), :]
            z_ref[pl.ds(trow, 1), :] = jnp.where(row >= 0, cur + acc, cur)
            return acc

        lax.fori_loop(0, 64, mrg, jnp.zeros((1, D), jnp.float32))
        o_ref[...] = jnp.dot(z_ref[pl.ds(0, N), :], w1_ref[...],
                             preferred_element_type=jnp.float32) + b1_ref[...]
        _stats(o_ref, mu_ref, va_ref, H)

    return pl.pallas_call(
        body,
        in_specs=[
            pl.BlockSpec(memory_space=pltpu.MemorySpace.VMEM),
            pl.BlockSpec(memory_space=pltpu.MemorySpace.VMEM),
            pl.BlockSpec(memory_space=pltpu.MemorySpace.SMEM),
            pl.BlockSpec(memory_space=pltpu.MemorySpace.VMEM),
            pl.BlockSpec(memory_space=pltpu.MemorySpace.VMEM),
            pl.BlockSpec(memory_space=pltpu.MemorySpace.SMEM),
            pl.BlockSpec(memory_space=pltpu.MemorySpace.SMEM),
            pl.BlockSpec(memory_space=pltpu.MemorySpace.SMEM),
            pl.BlockSpec(memory_space=pltpu.MemorySpace.SMEM),
        ],
        out_shape=[jax.ShapeDtypeStruct((N, H), jnp.float32),
                   jax.ShapeDtypeStruct((1, H), jnp.float32),
                   jax.ShapeDtypeStruct((1, H), jnp.float32)],
        scratch_shapes=[pltpu.VMEM((N + 8, D), jnp.float32)],
    )(h, parts, eps_l.reshape(1), w1, b1[None, :], start, use, wrow, core_of)


def _stage_b(y, mu, va, g1, be1, w2, b2):
    def body(y_ref, mu_ref, va_ref, g_ref, be_ref, w2_ref, b2_ref, o_ref,
             mu2_ref, va2_ref):
        z = ((y_ref[...] - mu_ref[...]) / jnp.sqrt(va_ref[...] + 1e-5)
             * g_ref[...] + be_ref[...])
        z = jnp.maximum(z, 0.0)
        o_ref[...] = jnp.dot(z, w2_ref[...],
                             preferred_element_type=jnp.float32) + b2_ref[...]
        _stats(o_ref, mu2_ref, va2_ref, D)

    return pl.pallas_call(
        body,
        out_shape=[jax.ShapeDtypeStruct((N, D), jnp.float32),
                   jax.ShapeDtypeStruct((1, D), jnp.float32),
                   jax.ShapeDtypeStruct((1, D), jnp.float32)],
    )(y, mu, va, g1[None, :], be1[None, :], w2, b2[None, :])


def _stage_c(z2, mu, va, bng, bnb, relu_last):
    def body(z_ref, mu_ref, va_ref, g_ref, be_ref, o_ref):
        z = ((z_ref[...] - mu_ref[...]) / jnp.sqrt(va_ref[...] + 1e-5)
             * g_ref[...] + be_ref[...])
        if relu_last:
            z = jnp.maximum(z, 0.0)
        o_ref[...] = z

    return pl.pallas_call(
        body,
        out_shape=jax.ShapeDtypeStruct((N, D), jnp.float32),
    )(z2, mu, va, bng[None, :], bnb[None, :])


def _pool_head(h, batch, head_w, head_b):
    def body(h_ref, b_ref, w_ref, bias_ref, o_ref):
        seg = b_ref[...]
        onehot = (seg == lax.broadcasted_iota(jnp.int32, (N, G), 1)
                  ).astype(jnp.float32)
        cnt = jnp.sum(onehot, axis=0)
        hs = lax.dot_general(onehot, h_ref[...], (((0,), (0,)), ((), ())),
                             preferred_element_type=jnp.float32,
                             precision=jax.lax.Precision.HIGHEST)
        hg = hs / jnp.maximum(cnt, 1.0)[:, None]
        o_ref[...] = (
            jnp.dot(hg, w_ref[...], preferred_element_type=jnp.float32)
            + bias_ref[...]
        )

    return pl.pallas_call(
        body,
        out_shape=jax.ShapeDtypeStruct((G, T), jnp.float32),
    )(h, batch[:, None], head_w, head_b[None, :])


def kernel(x, edge_index, edge_attr, batch, eps, edge_W, edge_b, W1, b1, g1,
           be1, W2, b2, bng, bnb, head_W, head_b):
    src = edge_index[0]
    dst = edge_index[1]
    (perm_p, srcs_p, tgt_p, keep_p, start, use, wrow,
     core_of) = _edge_plan(src, dst)
    h = x
    for l in range(L):
        e = _edge_embed(edge_attr, edge_W[l], edge_b[l])
        parts = _sc_edge(h, e, perm_p, srcs_p, tgt_p, keep_p)
        y, mu1, va1 = _stage_a(h, parts, eps[l], W1[l], b1[l], start, use,
                               wrow, core_of)
        z2, mu2, va2 = _stage_b(y, mu1, va1, g1[l], be1[l], W2[l], b2[l])
        h = _stage_c(z2, mu2, va2, bng[l], bnb[l], relu_last=(l < L - 1))
    return _pool_head(h, batch, head_W, head_b)
